# baseline (speedup 1.0000x reference)
import functools
import math

import jax
import jax.numpy as jnp
from jax import lax
from jax.experimental import pallas as pl
from jax.experimental.pallas import tpu as pltpu
from jax.experimental.pallas import tpu_sc as plsc

N = 10000
E = 160000
D = 128
C = 128

_NB = 10
_BN = N // _NB
_EB = 250
_BE = E // _EB

_NC = 2
_NS = 16
_NW = _NC * _NS
_CHUNK = 128
_NCHUNK = E // _CHUNK
_CPW = -(-_NCHUNK // _NW)
_U = 80
_NU = N // _U
_UPW = -(-_NU // _NS)


_EPAD = _NW * _CPW * _CHUNK
_EPW = _EPAD // _NW


def _sc_gather_body(src_ref, dst_ref, ta_ref, tv_ref, ts_ref,
                    ga_ref, gv_ref, gs_ref,
                    idx_d, idx_s, rows_a, rows_v, rows_s, *sems):
    c = lax.axis_index("c")
    s = lax.axis_index("s")
    wid = s * _NC + c
    base = wid * _EPW
    ga_sems, gv_sems, gs_sems = sems[0:2], sems[2:4], sems[4:6]
    wa_sems, wv_sems, ws_sems = sems[6:8], sems[8:10], sems[10:12]

    pltpu.sync_copy(dst_ref.at[pl.ds(base, _EPW)], idx_d)
    pltpu.sync_copy(src_ref.at[pl.ds(base, _EPW)], idx_s)

    def g_descs(b, k):
        i0 = k * _CHUNK
        return (
            pltpu.make_async_copy(
                ta_ref.at[idx_d.at[pl.ds(i0, _CHUNK)]], rows_a.at[b], ga_sems[b]),
            pltpu.make_async_copy(
                tv_ref.at[idx_d.at[pl.ds(i0, _CHUNK)]], rows_v.at[b], gv_sems[b]),
            pltpu.make_async_copy(
                ts_ref.at[idx_s.at[pl.ds(i0, _CHUNK)]], rows_s.at[b], gs_sems[b]),
        )

    def w_descs(b, k):
        off = base + k * _CHUNK
        return (
            pltpu.make_async_copy(
                rows_a.at[b], ga_ref.at[pl.ds(off, _CHUNK)], wa_sems[b]),
            pltpu.make_async_copy(
                rows_v.at[b], gv_ref.at[pl.ds(off, _CHUNK)], wv_sems[b]),
            pltpu.make_async_copy(
                rows_s.at[b], gs_ref.at[pl.ds(off, _CHUNK)], ws_sems[b]),
        )

    for b in (0, 1):
        for dsc in g_descs(b, b):
            dsc.start()

    def body(t, carry):
        for b in (0, 1):
            k = 2 * t + b
            for dsc in g_descs(b, k):
                dsc.wait()
            wds = w_descs(b, k)
            for dsc in wds:
                dsc.start()
            for dsc in wds:
                dsc.wait()

            @pl.when(k + 2 < _CPW)
            def _():
                for dsc in g_descs(b, k + 2):
                    dsc.start()

        return carry

    lax.fori_loop(0, _CPW // 2, body, 0)


def _sc_gather(src, dst, TDa, TDv, TS32):
    mesh = plsc.VectorSubcoreMesh(core_axis_name="c", subcore_axis_name="s")
    return pl.kernel(
        _sc_gather_body,
        out_type=[
            jax.ShapeDtypeStruct((_EPAD, C), jnp.int32),
            jax.ShapeDtypeStruct((_EPAD, C), jnp.float32),
            jax.ShapeDtypeStruct((_EPAD, C), jnp.int32),
        ],
        mesh=mesh,
        scratch_types=[
            pltpu.VMEM((_EPW,), jnp.int32),
            pltpu.VMEM((_EPW,), jnp.int32),
            pltpu.VMEM((2, _CHUNK, C), jnp.int32),
            pltpu.VMEM((2, _CHUNK, C), jnp.float32),
            pltpu.VMEM((2, _CHUNK, C), jnp.int32),
        ] + [pltpu.SemaphoreType.DMA] * 12,
    )(src, dst, TDa, TDv, TS32)


def _sc_scatter_body(z_ref, dst_ref, out_ref,
                     idx_v, z_v, stage_v, acc_shared, sem):
    c = lax.axis_index("c")
    s = lax.axis_index("s")
    wid = s * _NC + c

    def zbody(i, carry):
        r = i // 8
        l = i % 8
        stage_v[r, pl.ds(l * 16, 16)] = jnp.zeros((16,), jnp.float32)
        return carry

    lax.fori_loop(0, _U * 8, zbody, 0)

    def ubody(j, carry):
        u = s + j * _NS

        @pl.when(u < _NU)
        def _():
            pltpu.sync_copy(stage_v, acc_shared.at[pl.ds(u * _U, _U)])

        return carry

    lax.fori_loop(0, _UPW, ubody, 0)
    plsc.subcore_barrier()

    def body(j, carry):
        ci = wid + j * _NW

        @pl.when(ci < _NCHUNK)
        def _():
            off = ci * _CHUNK
            pltpu.sync_copy(dst_ref.at[pl.ds(off, _CHUNK)], idx_v)
            pltpu.sync_copy(z_ref.at[pl.ds(off, _CHUNK)], z_v)
            pltpu.sync_copy(z_v, acc_shared.at[idx_v], add=True)

        return carry

    lax.fori_loop(0, _CPW, body, 0)
    plsc.subcore_barrier()

    def dbody(j, carry):
        u = s + j * _NS

        @pl.when(u < _NU)
        def _():
            pltpu.sync_copy(acc_shared.at[pl.ds(u * _U, _U)], stage_v)
            pltpu.sync_copy(stage_v, out_ref.at[c, pl.ds(u * _U, _U)])

        return carry

    lax.fori_loop(0, _UPW, dbody, 0)


def _sc_scatter(z, dst):
    mesh = plsc.VectorSubcoreMesh(core_axis_name="c", subcore_axis_name="s")
    return pl.kernel(
        _sc_scatter_body,
        out_type=jax.ShapeDtypeStruct((_NC, N, C), jnp.float32),
        mesh=mesh,
        scratch_types=[
            pltpu.VMEM((_CHUNK,), jnp.int32),
            pltpu.VMEM((_CHUNK, C), jnp.float32),
            pltpu.VMEM((_U, C), jnp.float32),
            pltpu.VMEM_SHARED((N, C), jnp.float32),
            pltpu.SemaphoreType.DMA,
        ],
    )(z, dst)


def _prologue_body(x_ref, wq_ref, bq_ref, wk_ref, bk_ref, wv_ref, bv_ref,
                   td_ref, tv_ref, ts_ref):
    x = x_ref[...]
    q = jnp.dot(x, wq_ref[...], preferred_element_type=jnp.float32) + bq_ref[...]
    k = jnp.dot(x, wk_ref[...], preferred_element_type=jnp.float32) + bk_ref[...]
    v = jnp.dot(x, wv_ref[...], preferred_element_type=jnp.float32) + bv_ref[...]

    def bits(a):
        return jax.lax.bitcast_convert_type(
            a.astype(jnp.bfloat16).astype(jnp.float32), jnp.int32)

    def pack(lo, hi):
        return jax.lax.shift_right_logical(bits(lo), 16) | bits(hi)

    td_ref[...] = pack(q, q * k)
    tv_ref[...] = v
    ts_ref[...] = pack(k, v)


def _prologue(x, W_query, b_query, W_key, b_key, W_value, b_value, interpret=False):
    full = lambda shape: pl.BlockSpec(shape, lambda i: (0, 0))
    return pl.pallas_call(
        _prologue_body,
        grid=(_NB,),
        in_specs=[
            pl.BlockSpec((_BN, D), lambda i: (i, 0)),
            full((D, C)), full((1, C)),
            full((D, C)), full((1, C)),
            full((D, C)), full((1, C)),
        ],
        out_specs=[
            pl.BlockSpec((_BN, C), lambda i: (i, 0)),
            pl.BlockSpec((_BN, C), lambda i: (i, 0)),
            pl.BlockSpec((_BN, C), lambda i: (i, 0)),
        ],
        out_shape=[
            jax.ShapeDtypeStruct((N, C), jnp.int32),
            jax.ShapeDtypeStruct((N, C), jnp.float32),
            jax.ShapeDtypeStruct((N, C), jnp.int32),
        ],
        interpret=interpret,
    )(x, W_query, b_query.reshape(1, C), W_key, b_key.reshape(1, C),
      W_value, b_value.reshape(1, C))


def _edge_body(ea_ref, ga_ref, gv_ref, gs_ref, wedge_ref,
               wmu1_ref, wmu2_ref, wmu3_ref, bmu_ref,
               wmsg1_ref, wmsg2_ref, wmsg3_ref, bmsg_ref,
               lag_ref, lab_ref, lmg_ref, lmb_ref, z_ref):
    ea = ea_ref[...]
    e = jnp.dot(ea, wedge_ref[...], preferred_element_type=jnp.float32)
    gai = ga_ref[...]
    gsi = gs_ref[...]
    f32 = jnp.float32
    bf16 = jnp.bfloat16
    q_i = jax.lax.bitcast_convert_type(jax.lax.shift_left(gai, 16), f32)
    qk_i = jax.lax.bitcast_convert_type(gai & jnp.int32(-65536), f32)
    k_j = jax.lax.bitcast_convert_type(jax.lax.shift_left(gsi, 16), f32)
    v_j = jax.lax.bitcast_convert_type(gsi & jnp.int32(-65536), f32)
    v_i = gv_ref[...]
    scale = 1.0 / math.sqrt(3.0 * C)
    a1 = qk_i * scale
    a2 = (q_i * k_j) * scale
    a3 = (q_i * e) * scale
    inv = 1.0 / (3.0 * C)
    m = (jnp.sum(a1, axis=1, keepdims=True) + jnp.sum(a2, axis=1, keepdims=True)
         + jnp.sum(a3, axis=1, keepdims=True)) * inv
    sq = (jnp.sum(a1 * a1, axis=1, keepdims=True)
          + jnp.sum(a2 * a2, axis=1, keepdims=True)
          + jnp.sum(a3 * a3, axis=1, keepdims=True)) * inv
    rstd = jax.lax.rsqrt(sq - m * m + 1e-5)
    lag = lag_ref[...]
    lab = lab_ref[...]
    g1 = jax.nn.sigmoid((a1 - m) * rstd * lag[:, 0:C] + lab[:, 0:C])
    g2 = jax.nn.sigmoid((a2 - m) * rstd * lag[:, C:2 * C] + lab[:, C:2 * C])
    g3 = jax.nn.sigmoid((a3 - m) * rstd * lag[:, 2 * C:3 * C] + lab[:, 2 * C:3 * C])
    upd = (jnp.dot(v_i.astype(bf16), wmu1_ref[...], preferred_element_type=f32)
           + jnp.dot(v_j.astype(bf16), wmu2_ref[...], preferred_element_type=f32)
           + jnp.dot(e.astype(bf16), wmu3_ref[...], preferred_element_type=f32)
           + bmu_ref[...])
    t1 = (upd[:, 0:C] * g1).astype(bf16)
    t2 = (upd[:, C:2 * C] * g2).astype(bf16)
    t3 = (upd[:, 2 * C:3 * C] * g3).astype(bf16)
    z = (jnp.dot(t1, wmsg1_ref[...], preferred_element_type=f32)
         + jnp.dot(t2, wmsg2_ref[...], preferred_element_type=f32)
         + jnp.dot(t3, wmsg3_ref[...], preferred_element_type=f32)
         + bmsg_ref[...])
    zm = jnp.mean(z, axis=1, keepdims=True)
    zv = jnp.mean(z * z, axis=1, keepdims=True) - zm * zm
    z_ref[...] = (z - zm) * jax.lax.rsqrt(zv + 1e-5) * lmg_ref[...] + lmb_ref[...]


def _edge_stage(edge_attr, G_a, G_v, G_s, W_edge, W_msg_update, b_msg_update,
                W_msg, b_msg, ln_alpha_g, ln_alpha_b, ln_msg_g, ln_msg_b,
                interpret=False):
    full = lambda shape: pl.BlockSpec(shape, lambda i: (0, 0))
    return pl.pallas_call(
        _edge_body,
        grid=(_EB,),
        in_specs=[
            pl.BlockSpec((_BE, 16), lambda i: (i, 0)),
            pl.BlockSpec((_BE, C), lambda i: (i, 0)),
            pl.BlockSpec((_BE, C), lambda i: (i, 0)),
            pl.BlockSpec((_BE, C), lambda i: (i, 0)),
            full((16, C)),
            full((C, 3 * C)), full((C, 3 * C)), full((C, 3 * C)),
            full((1, 3 * C)),
            full((C, C)), full((C, C)), full((C, C)),
            full((1, C)),
            full((1, 3 * C)), full((1, 3 * C)),
            full((1, C)), full((1, C)),
        ],
        out_specs=pl.BlockSpec((_BE, C), lambda i: (i, 0)),
        out_shape=jax.ShapeDtypeStruct((E, C), jnp.float32),
        interpret=interpret,
    )(edge_attr, G_a, G_v, G_s, W_edge,
      W_msg_update[0:C].astype(jnp.bfloat16),
      W_msg_update[C:2 * C].astype(jnp.bfloat16),
      W_msg_update[2 * C:3 * C].astype(jnp.bfloat16),
      b_msg_update.reshape(1, 3 * C),
      W_msg[0:C].astype(jnp.bfloat16),
      W_msg[C:2 * C].astype(jnp.bfloat16),
      W_msg[2 * C:3 * C].astype(jnp.bfloat16),
      b_msg.reshape(1, C),
      ln_alpha_g.reshape(1, 3 * C), ln_alpha_b.reshape(1, 3 * C),
      ln_msg_g.reshape(1, C), ln_msg_b.reshape(1, C))


def _epi1_body(agg0_ref, agg1_ref, wc_ref, bc_ref, out1_ref, ssum_ref, ssq_ref):
    i = pl.program_id(0)
    agg = agg0_ref[...] + agg1_ref[...]
    o = jnp.dot(agg, wc_ref[...], preferred_element_type=jnp.float32) + bc_ref[...]
    out1_ref[...] = o
    s = jnp.sum(o, axis=0, keepdims=True)
    sq = jnp.sum(o * o, axis=0, keepdims=True)

    @pl.when(i == 0)
    def _():
        ssum_ref[...] = jnp.zeros_like(ssum_ref)
        ssq_ref[...] = jnp.zeros_like(ssq_ref)

    ssum_ref[...] += s
    ssq_ref[...] += sq


def _epi1(agg0, agg1, W_concate, b_concate, interpret=False):
    full = lambda shape: pl.BlockSpec(shape, lambda i: (0, 0))
    return pl.pallas_call(
        _epi1_body,
        grid=(_NB,),
        in_specs=[
            pl.BlockSpec((_BN, C), lambda i: (i, 0)),
            pl.BlockSpec((_BN, C), lambda i: (i, 0)),
            full((C, C)), full((1, C)),
        ],
        out_specs=[
            pl.BlockSpec((_BN, C), lambda i: (i, 0)),
            full((1, C)), full((1, C)),
        ],
        out_shape=[
            jax.ShapeDtypeStruct((N, C), jnp.float32),
            jax.ShapeDtypeStruct((1, C), jnp.float32),
            jax.ShapeDtypeStruct((1, C), jnp.float32),
        ],
        interpret=interpret,
    )(agg0, agg1, W_concate, b_concate.reshape(1, C))


def _epi2_body(out1_ref, ssum_ref, ssq_ref, x_ref, wskip_ref, bskip_ref,
               bng_ref, bnb_ref, out_ref):
    o = out1_ref[...]
    mean = ssum_ref[...] * (1.0 / N)
    var = ssq_ref[...] * (1.0 / N) - mean * mean
    o = bng_ref[...] * (o - mean) * jax.lax.rsqrt(var + 1e-5) + bnb_ref[...]
    o = o * jax.nn.sigmoid(o)
    skip = jnp.dot(x_ref[...], wskip_ref[...], preferred_element_type=jnp.float32) + bskip_ref[...]
    out_ref[...] = o + skip


def _epi2(out1, ssum, ssq, x, W_skip, b_skip, bn_g, bn_b, interpret=False):
    full = lambda shape: pl.BlockSpec(shape, lambda i: (0, 0))
    return pl.pallas_call(
        _epi2_body,
        grid=(_NB,),
        in_specs=[
            pl.BlockSpec((_BN, C), lambda i: (i, 0)),
            full((1, C)), full((1, C)),
            pl.BlockSpec((_BN, D), lambda i: (i, 0)),
            full((D, C)), full((1, C)),
            full((1, C)), full((1, C)),
        ],
        out_specs=pl.BlockSpec((_BN, C), lambda i: (i, 0)),
        out_shape=jax.ShapeDtypeStruct((N, C), jnp.float32),
        interpret=interpret,
    )(out1, ssum, ssq, x, W_skip, b_skip.reshape(1, C),
      bn_g.reshape(1, C), bn_b.reshape(1, C))


def kernel(x, edge_index, edge_attr, W_query, b_query, W_key, b_key,
           W_value, b_value, W_edge, W_msg_update, b_msg_update, W_msg,
           b_msg, ln_msg_g, ln_msg_b, ln_alpha_g, ln_alpha_b, W_concate,
           b_concate, bn_g, bn_b, W_skip, b_skip):
    src = edge_index[0]
    dst = edge_index[1]
    pad = jnp.zeros((_EPAD - E,), jnp.int32)
    TDa, TV, TS32 = _prologue(x, W_query, b_query, W_key, b_key,
                              W_value, b_value)
    G_a, G_v, G_s = _sc_gather(jnp.concatenate([src, pad]),
                               jnp.concatenate([dst, pad]), TDa, TV, TS32)
    z = _edge_stage(edge_attr, G_a, G_v, G_s, W_edge, W_msg_update,
                    b_msg_update, W_msg, b_msg, ln_alpha_g, ln_alpha_b,
                    ln_msg_g, ln_msg_b)
    parts = _sc_scatter(z, dst)
    out1, ssum, ssq = _epi1(parts[0], parts[1], W_concate, b_concate)
    return _epi2(out1, ssum, ssq, x, W_skip, b_skip, bn_g, bn_b)


# speedup vs baseline: 1.0263x; 1.0263x over previous
import functools
import math

import jax
import jax.numpy as jnp
from jax import lax
from jax.experimental import pallas as pl
from jax.experimental.pallas import tpu as pltpu
from jax.experimental.pallas import tpu_sc as plsc

N = 10000
E = 160000
D = 128
C = 128

_NB = 10
_BN = N // _NB
_EB = 250
_BE = E // _EB

_NC = 2
_NS = 16
_NW = _NC * _NS
_CHUNK = 128
_NCHUNK = E // _CHUNK
_CPW = -(-_NCHUNK // _NW)
_U = 80
_NU = N // _U
_UPW = -(-_NU // _NS)


_EPAD = _NW * _CPW * _CHUNK
_EPW = _EPAD // _NW


def _sc_gather_body(src_ref, dst_ref, td_ref, ts_ref,
                    gd_ref, gs_ref,
                    idx_d, idx_s, rows_d, rows_s, *sems):
    c = lax.axis_index("c")
    s = lax.axis_index("s")
    wid = s * _NC + c
    base = wid * _EPW
    gd_sems, gs_sems = sems[0:2], sems[2:4]
    wd_sems, ws_sems = sems[4:6], sems[6:8]

    pltpu.sync_copy(dst_ref.at[pl.ds(base, _EPW)], idx_d)
    pltpu.sync_copy(src_ref.at[pl.ds(base, _EPW)], idx_s)

    def g_descs(b, k):
        i0 = k * _CHUNK
        return (
            pltpu.make_async_copy(
                td_ref.at[idx_d.at[pl.ds(i0, _CHUNK)]], rows_d.at[b], gd_sems[b]),
            pltpu.make_async_copy(
                ts_ref.at[idx_s.at[pl.ds(i0, _CHUNK)]], rows_s.at[b], gs_sems[b]),
        )

    def w_descs(b, k):
        off = base + k * _CHUNK
        return (
            pltpu.make_async_copy(
                rows_d.at[b], gd_ref.at[pl.ds(off, _CHUNK)], wd_sems[b]),
            pltpu.make_async_copy(
                rows_s.at[b], gs_ref.at[pl.ds(off, _CHUNK)], ws_sems[b]),
        )

    for b in (0, 1):
        for dsc in g_descs(b, b):
            dsc.start()

    def body(t, carry):
        for b in (0, 1):
            k = 2 * t + b
            for dsc in g_descs(b, k):
                dsc.wait()
            wds = w_descs(b, k)
            for dsc in wds:
                dsc.start()
            for dsc in wds:
                dsc.wait()

            @pl.when(k + 2 < _CPW)
            def _():
                for dsc in g_descs(b, k + 2):
                    dsc.start()

        return carry

    lax.fori_loop(0, _CPW // 2, body, 0)


def _sc_gather(src, dst, TDdv, TS32):
    mesh = plsc.VectorSubcoreMesh(core_axis_name="c", subcore_axis_name="s")
    return pl.kernel(
        _sc_gather_body,
        out_type=[
            jax.ShapeDtypeStruct((_EPAD, 2 * C), jnp.int32),
            jax.ShapeDtypeStruct((_EPAD, C), jnp.int32),
        ],
        mesh=mesh,
        scratch_types=[
            pltpu.VMEM((_EPW,), jnp.int32),
            pltpu.VMEM((_EPW,), jnp.int32),
            pltpu.VMEM((2, _CHUNK, 2 * C), jnp.int32),
            pltpu.VMEM((2, _CHUNK, C), jnp.int32),
        ] + [pltpu.SemaphoreType.DMA] * 8,
    )(src, dst, TDdv, TS32)


def _sc_scatter_body(z_ref, dst_ref, out_ref,
                     idx_v, z_v, stage_v, acc_shared, sem):
    c = lax.axis_index("c")
    s = lax.axis_index("s")
    wid = s * _NC + c

    def zbody(i, carry):
        r = i // 8
        l = i % 8
        stage_v[r, pl.ds(l * 16, 16)] = jnp.zeros((16,), jnp.float32)
        return carry

    lax.fori_loop(0, _U * 8, zbody, 0)

    def ubody(j, carry):
        u = s + j * _NS

        @pl.when(u < _NU)
        def _():
            pltpu.sync_copy(stage_v, acc_shared.at[pl.ds(u * _U, _U)])

        return carry

    lax.fori_loop(0, _UPW, ubody, 0)
    plsc.subcore_barrier()

    def body(j, carry):
        ci = wid + j * _NW

        @pl.when(ci < _NCHUNK)
        def _():
            off = ci * _CHUNK
            pltpu.sync_copy(dst_ref.at[pl.ds(off, _CHUNK)], idx_v)
            pltpu.sync_copy(z_ref.at[pl.ds(off, _CHUNK)], z_v)
            pltpu.sync_copy(z_v, acc_shared.at[idx_v], add=True)

        return carry

    lax.fori_loop(0, _CPW, body, 0)
    plsc.subcore_barrier()

    def dbody(j, carry):
        u = s + j * _NS

        @pl.when(u < _NU)
        def _():
            pltpu.sync_copy(acc_shared.at[pl.ds(u * _U, _U)], stage_v)
            pltpu.sync_copy(stage_v, out_ref.at[c, pl.ds(u * _U, _U)])

        return carry

    lax.fori_loop(0, _UPW, dbody, 0)


def _sc_scatter(z, dst):
    mesh = plsc.VectorSubcoreMesh(core_axis_name="c", subcore_axis_name="s")
    return pl.kernel(
        _sc_scatter_body,
        out_type=jax.ShapeDtypeStruct((_NC, N, C), jnp.float32),
        mesh=mesh,
        scratch_types=[
            pltpu.VMEM((_CHUNK,), jnp.int32),
            pltpu.VMEM((_CHUNK, C), jnp.float32),
            pltpu.VMEM((_U, C), jnp.float32),
            pltpu.VMEM_SHARED((N, C), jnp.float32),
            pltpu.SemaphoreType.DMA,
        ],
    )(z, dst)


def _prologue_body(x_ref, wq_ref, bq_ref, wk_ref, bk_ref, wv_ref, bv_ref,
                   td_ref, ts_ref):
    x = x_ref[...]
    q = jnp.dot(x, wq_ref[...], preferred_element_type=jnp.float32) + bq_ref[...]
    k = jnp.dot(x, wk_ref[...], preferred_element_type=jnp.float32) + bk_ref[...]
    v = jnp.dot(x, wv_ref[...], preferred_element_type=jnp.float32) + bv_ref[...]

    def bits(a):
        return jax.lax.bitcast_convert_type(
            a.astype(jnp.bfloat16).astype(jnp.float32), jnp.int32)

    def pack(lo, hi):
        return jax.lax.shift_right_logical(bits(lo), 16) | bits(hi)

    td_ref[...] = jnp.concatenate(
        [pack(q, q * k), jax.lax.bitcast_convert_type(v, jnp.int32)], axis=1)
    ts_ref[...] = pack(k, v)


def _prologue(x, W_query, b_query, W_key, b_key, W_value, b_value, interpret=False):
    full = lambda shape: pl.BlockSpec(shape, lambda i: (0, 0))
    return pl.pallas_call(
        _prologue_body,
        grid=(_NB,),
        in_specs=[
            pl.BlockSpec((_BN, D), lambda i: (i, 0)),
            full((D, C)), full((1, C)),
            full((D, C)), full((1, C)),
            full((D, C)), full((1, C)),
        ],
        out_specs=[
            pl.BlockSpec((_BN, 2 * C), lambda i: (i, 0)),
            pl.BlockSpec((_BN, C), lambda i: (i, 0)),
        ],
        out_shape=[
            jax.ShapeDtypeStruct((N, 2 * C), jnp.int32),
            jax.ShapeDtypeStruct((N, C), jnp.int32),
        ],
        interpret=interpret,
    )(x, W_query, b_query.reshape(1, C), W_key, b_key.reshape(1, C),
      W_value, b_value.reshape(1, C))


def _edge_body(ea_ref, gd_ref, gs_ref, wedge_ref,
               wmu1_ref, wmu2_ref, wmu3_ref, bmu_ref,
               wmsg1_ref, wmsg2_ref, wmsg3_ref, bmsg_ref,
               lag_ref, lab_ref, lmg_ref, lmb_ref, z_ref):
    ea = ea_ref[...]
    e = jnp.dot(ea, wedge_ref[...], preferred_element_type=jnp.float32)
    gdi = gd_ref[...]
    gai = gdi[:, 0:C]
    gsi = gs_ref[...]
    f32 = jnp.float32
    bf16 = jnp.bfloat16
    q_i = jax.lax.bitcast_convert_type(jax.lax.shift_left(gai, 16), f32)
    qk_i = jax.lax.bitcast_convert_type(gai & jnp.int32(-65536), f32)
    k_j = jax.lax.bitcast_convert_type(jax.lax.shift_left(gsi, 16), f32)
    v_j = jax.lax.bitcast_convert_type(gsi & jnp.int32(-65536), f32)
    v_i = jax.lax.bitcast_convert_type(gdi[:, C:2 * C], f32)
    scale = 1.0 / math.sqrt(3.0 * C)
    a1 = qk_i * scale
    a2 = (q_i * k_j) * scale
    a3 = (q_i * e) * scale
    inv = 1.0 / (3.0 * C)
    m = (jnp.sum(a1, axis=1, keepdims=True) + jnp.sum(a2, axis=1, keepdims=True)
         + jnp.sum(a3, axis=1, keepdims=True)) * inv
    sq = (jnp.sum(a1 * a1, axis=1, keepdims=True)
          + jnp.sum(a2 * a2, axis=1, keepdims=True)
          + jnp.sum(a3 * a3, axis=1, keepdims=True)) * inv
    rstd = jax.lax.rsqrt(sq - m * m + 1e-5)
    lag = lag_ref[...]
    lab = lab_ref[...]
    g1 = jax.nn.sigmoid((a1 - m) * rstd * lag[:, 0:C] + lab[:, 0:C])
    g2 = jax.nn.sigmoid((a2 - m) * rstd * lag[:, C:2 * C] + lab[:, C:2 * C])
    g3 = jax.nn.sigmoid((a3 - m) * rstd * lag[:, 2 * C:3 * C] + lab[:, 2 * C:3 * C])
    upd = (jnp.dot(v_i.astype(bf16), wmu1_ref[...], preferred_element_type=f32)
           + jnp.dot(v_j.astype(bf16), wmu2_ref[...], preferred_element_type=f32)
           + jnp.dot(e.astype(bf16), wmu3_ref[...], preferred_element_type=f32)
           + bmu_ref[...])
    t1 = (upd[:, 0:C] * g1).astype(bf16)
    t2 = (upd[:, C:2 * C] * g2).astype(bf16)
    t3 = (upd[:, 2 * C:3 * C] * g3).astype(bf16)
    z = (jnp.dot(t1, wmsg1_ref[...], preferred_element_type=f32)
         + jnp.dot(t2, wmsg2_ref[...], preferred_element_type=f32)
         + jnp.dot(t3, wmsg3_ref[...], preferred_element_type=f32)
         + bmsg_ref[...])
    zm = jnp.mean(z, axis=1, keepdims=True)
    zv = jnp.mean(z * z, axis=1, keepdims=True) - zm * zm
    z_ref[...] = (z - zm) * jax.lax.rsqrt(zv + 1e-5) * lmg_ref[...] + lmb_ref[...]


def _edge_stage(edge_attr, G_dv, G_s, W_edge, W_msg_update, b_msg_update,
                W_msg, b_msg, ln_alpha_g, ln_alpha_b, ln_msg_g, ln_msg_b,
                interpret=False):
    full = lambda shape: pl.BlockSpec(shape, lambda i: (0, 0))
    return pl.pallas_call(
        _edge_body,
        grid=(_EB,),
        in_specs=[
            pl.BlockSpec((_BE, 16), lambda i: (i, 0)),
            pl.BlockSpec((_BE, 2 * C), lambda i: (i, 0)),
            pl.BlockSpec((_BE, C), lambda i: (i, 0)),
            full((16, C)),
            full((C, 3 * C)), full((C, 3 * C)), full((C, 3 * C)),
            full((1, 3 * C)),
            full((C, C)), full((C, C)), full((C, C)),
            full((1, C)),
            full((1, 3 * C)), full((1, 3 * C)),
            full((1, C)), full((1, C)),
        ],
        out_specs=pl.BlockSpec((_BE, C), lambda i: (i, 0)),
        out_shape=jax.ShapeDtypeStruct((E, C), jnp.float32),
        interpret=interpret,
    )(edge_attr, G_dv, G_s, W_edge,
      W_msg_update[0:C].astype(jnp.bfloat16),
      W_msg_update[C:2 * C].astype(jnp.bfloat16),
      W_msg_update[2 * C:3 * C].astype(jnp.bfloat16),
      b_msg_update.reshape(1, 3 * C),
      W_msg[0:C].astype(jnp.bfloat16),
      W_msg[C:2 * C].astype(jnp.bfloat16),
      W_msg[2 * C:3 * C].astype(jnp.bfloat16),
      b_msg.reshape(1, C),
      ln_alpha_g.reshape(1, 3 * C), ln_alpha_b.reshape(1, 3 * C),
      ln_msg_g.reshape(1, C), ln_msg_b.reshape(1, C))


def _epi1_body(agg0_ref, agg1_ref, wc_ref, bc_ref, out1_ref, ssum_ref, ssq_ref):
    i = pl.program_id(0)
    agg = agg0_ref[...] + agg1_ref[...]
    o = jnp.dot(agg, wc_ref[...], preferred_element_type=jnp.float32) + bc_ref[...]
    out1_ref[...] = o
    s = jnp.sum(o, axis=0, keepdims=True)
    sq = jnp.sum(o * o, axis=0, keepdims=True)

    @pl.when(i == 0)
    def _():
        ssum_ref[...] = jnp.zeros_like(ssum_ref)
        ssq_ref[...] = jnp.zeros_like(ssq_ref)

    ssum_ref[...] += s
    ssq_ref[...] += sq


def _epi1(agg0, agg1, W_concate, b_concate, interpret=False):
    full = lambda shape: pl.BlockSpec(shape, lambda i: (0, 0))
    return pl.pallas_call(
        _epi1_body,
        grid=(_NB,),
        in_specs=[
            pl.BlockSpec((_BN, C), lambda i: (i, 0)),
            pl.BlockSpec((_BN, C), lambda i: (i, 0)),
            full((C, C)), full((1, C)),
        ],
        out_specs=[
            pl.BlockSpec((_BN, C), lambda i: (i, 0)),
            full((1, C)), full((1, C)),
        ],
        out_shape=[
            jax.ShapeDtypeStruct((N, C), jnp.float32),
            jax.ShapeDtypeStruct((1, C), jnp.float32),
            jax.ShapeDtypeStruct((1, C), jnp.float32),
        ],
        interpret=interpret,
    )(agg0, agg1, W_concate, b_concate.reshape(1, C))


def _epi2_body(out1_ref, ssum_ref, ssq_ref, x_ref, wskip_ref, bskip_ref,
               bng_ref, bnb_ref, out_ref):
    o = out1_ref[...]
    mean = ssum_ref[...] * (1.0 / N)
    var = ssq_ref[...] * (1.0 / N) - mean * mean
    o = bng_ref[...] * (o - mean) * jax.lax.rsqrt(var + 1e-5) + bnb_ref[...]
    o = o * jax.nn.sigmoid(o)
    skip = jnp.dot(x_ref[...], wskip_ref[...], preferred_element_type=jnp.float32) + bskip_ref[...]
    out_ref[...] = o + skip


def _epi2(out1, ssum, ssq, x, W_skip, b_skip, bn_g, bn_b, interpret=False):
    full = lambda shape: pl.BlockSpec(shape, lambda i: (0, 0))
    return pl.pallas_call(
        _epi2_body,
        grid=(_NB,),
        in_specs=[
            pl.BlockSpec((_BN, C), lambda i: (i, 0)),
            full((1, C)), full((1, C)),
            pl.BlockSpec((_BN, D), lambda i: (i, 0)),
            full((D, C)), full((1, C)),
            full((1, C)), full((1, C)),
        ],
        out_specs=pl.BlockSpec((_BN, C), lambda i: (i, 0)),
        out_shape=jax.ShapeDtypeStruct((N, C), jnp.float32),
        interpret=interpret,
    )(out1, ssum, ssq, x, W_skip, b_skip.reshape(1, C),
      bn_g.reshape(1, C), bn_b.reshape(1, C))


def kernel(x, edge_index, edge_attr, W_query, b_query, W_key, b_key,
           W_value, b_value, W_edge, W_msg_update, b_msg_update, W_msg,
           b_msg, ln_msg_g, ln_msg_b, ln_alpha_g, ln_alpha_b, W_concate,
           b_concate, bn_g, bn_b, W_skip, b_skip):
    src = edge_index[0]
    dst = edge_index[1]
    pad = jnp.zeros((_EPAD - E,), jnp.int32)
    TDdv, TS32 = _prologue(x, W_query, b_query, W_key, b_key,
                           W_value, b_value)
    G_dv, G_s = _sc_gather(jnp.concatenate([src, pad]),
                           jnp.concatenate([dst, pad]), TDdv, TS32)
    z = _edge_stage(edge_attr, G_dv, G_s, W_edge, W_msg_update,
                    b_msg_update, W_msg, b_msg, ln_alpha_g, ln_alpha_b,
                    ln_msg_g, ln_msg_b)
    parts = _sc_scatter(z, dst)
    out1, ssum, ssq = _epi1(parts[0], parts[1], W_concate, b_concate)
    return _epi2(out1, ssum, ssq, x, W_skip, b_skip, bn_g, bn_b)


# speedup vs baseline: 1.1123x; 1.0839x over previous
import functools
import math

import jax
import jax.numpy as jnp
from jax import lax
from jax.experimental import pallas as pl
from jax.experimental.pallas import tpu as pltpu
from jax.experimental.pallas import tpu_sc as plsc

N = 10000
E = 160000
D = 128
C = 128

_NB = 10
_BN = N // _NB
_EB = 250
_BE = E // _EB

_NC = 2
_NS = 16
_NW = _NC * _NS
_CHUNK = 128
_NCHUNK = E // _CHUNK
_CPW = -(-_NCHUNK // _NW)
_U = 80
_NU = N // _U
_UPW = -(-_NU // _NS)


_EPAD = _NW * _CPW * _CHUNK
_EPW = _EPAD // _NW


_NSPLIT = 2
_EHALF = _EPAD // _NSPLIT
_EPW_H = _EHALF // _NW
_CPW_H = _EPW_H // _CHUNK


def _make_gather_body(base0):
    def _sc_gather_body(src_ref, dst_ref, td_ref, ts_ref,
                        gd_ref, gs_ref,
                        idx_d, idx_s, rows_d, rows_s, *sems):
        c = lax.axis_index("c")
        s = lax.axis_index("s")
        wid = s * _NC + c
        wbase = wid * _EPW_H
        gd_sems, gs_sems = sems[0:2], sems[2:4]
        wd_sems, ws_sems = sems[4:6], sems[6:8]

        pltpu.sync_copy(dst_ref.at[pl.ds(base0 + wbase, _EPW_H)], idx_d)
        pltpu.sync_copy(src_ref.at[pl.ds(base0 + wbase, _EPW_H)], idx_s)

        def g_descs(b, k):
            i0 = k * _CHUNK
            return (
                pltpu.make_async_copy(
                    td_ref.at[idx_d.at[pl.ds(i0, _CHUNK)]], rows_d.at[b],
                    gd_sems[b]),
                pltpu.make_async_copy(
                    ts_ref.at[idx_s.at[pl.ds(i0, _CHUNK)]], rows_s.at[b],
                    gs_sems[b]),
            )

        def w_descs(b, k):
            off = wbase + k * _CHUNK
            return (
                pltpu.make_async_copy(
                    rows_d.at[b], gd_ref.at[pl.ds(off, _CHUNK)], wd_sems[b]),
                pltpu.make_async_copy(
                    rows_s.at[b], gs_ref.at[pl.ds(off, _CHUNK)], ws_sems[b]),
            )

        for b in (0, 1):
            for dsc in g_descs(b, b):
                dsc.start()

        def body(t, carry):
            for b in (0, 1):
                k = 2 * t + b
                for dsc in g_descs(b, k):
                    dsc.wait()
                wds = w_descs(b, k)
                for dsc in wds:
                    dsc.start()
                for dsc in wds:
                    dsc.wait()

                @pl.when(k + 2 < _CPW_H)
                def _():
                    for dsc in g_descs(b, k + 2):
                        dsc.start()

            return carry

        lax.fori_loop(0, _CPW_H // 2, body, 0)

    return _sc_gather_body


def _sc_gather(src, dst, TDdv, TS32, half):
    mesh = plsc.VectorSubcoreMesh(core_axis_name="c", subcore_axis_name="s")
    return pl.kernel(
        _make_gather_body(half * _EHALF),
        out_type=[
            jax.ShapeDtypeStruct((_EHALF, 2 * C), jnp.int32),
            jax.ShapeDtypeStruct((_EHALF, C), jnp.int32),
        ],
        mesh=mesh,
        scratch_types=[
            pltpu.VMEM((_EPW_H,), jnp.int32),
            pltpu.VMEM((_EPW_H,), jnp.int32),
            pltpu.VMEM((2, _CHUNK, 2 * C), jnp.int32),
            pltpu.VMEM((2, _CHUNK, C), jnp.int32),
        ] + [pltpu.SemaphoreType.DMA] * 8,
    )(src, dst, TDdv, TS32)


def _make_scatter_body(base0, nchunk):
    cpw = -(-nchunk // _NW)

    def _sc_scatter_body(z_ref, dst_ref, out_ref,
                         idx_v, z_v, stage_v, acc_shared, sem):
        c = lax.axis_index("c")
        s = lax.axis_index("s")
        wid = s * _NC + c

        def zbody(i, carry):
            r = i // 8
            l = i % 8
            stage_v[r, pl.ds(l * 16, 16)] = jnp.zeros((16,), jnp.float32)
            return carry

        lax.fori_loop(0, _U * 8, zbody, 0)

        def ubody(j, carry):
            u = s + j * _NS

            @pl.when(u < _NU)
            def _():
                pltpu.sync_copy(stage_v, acc_shared.at[pl.ds(u * _U, _U)])

            return carry

        lax.fori_loop(0, _UPW, ubody, 0)
        plsc.subcore_barrier()

        def body(j, carry):
            ci = wid + j * _NW

            @pl.when(ci < nchunk)
            def _():
                off = ci * _CHUNK
                pltpu.sync_copy(dst_ref.at[pl.ds(base0 + off, _CHUNK)], idx_v)
                pltpu.sync_copy(z_ref.at[pl.ds(off, _CHUNK)], z_v)
                pltpu.sync_copy(z_v, acc_shared.at[idx_v], add=True)

            return carry

        lax.fori_loop(0, cpw, body, 0)
        plsc.subcore_barrier()

        def dbody(j, carry):
            u = s + j * _NS

            @pl.when(u < _NU)
            def _():
                pltpu.sync_copy(acc_shared.at[pl.ds(u * _U, _U)], stage_v)
                pltpu.sync_copy(stage_v, out_ref.at[c, pl.ds(u * _U, _U)])

            return carry

        lax.fori_loop(0, _UPW, dbody, 0)

    return _sc_scatter_body


def _sc_scatter(z, dst, base0, nchunk):
    mesh = plsc.VectorSubcoreMesh(core_axis_name="c", subcore_axis_name="s")
    return pl.kernel(
        _make_scatter_body(base0, nchunk),
        out_type=jax.ShapeDtypeStruct((_NC, N, C), jnp.float32),
        mesh=mesh,
        scratch_types=[
            pltpu.VMEM((_CHUNK,), jnp.int32),
            pltpu.VMEM((_CHUNK, C), jnp.float32),
            pltpu.VMEM((_U, C), jnp.float32),
            pltpu.VMEM_SHARED((N, C), jnp.float32),
            pltpu.SemaphoreType.DMA,
        ],
    )(z, dst)


def _prologue_body(x_ref, wq_ref, bq_ref, wk_ref, bk_ref, wv_ref, bv_ref,
                   td_ref, ts_ref):
    x = x_ref[...]
    q = jnp.dot(x, wq_ref[...], preferred_element_type=jnp.float32) + bq_ref[...]
    k = jnp.dot(x, wk_ref[...], preferred_element_type=jnp.float32) + bk_ref[...]
    v = jnp.dot(x, wv_ref[...], preferred_element_type=jnp.float32) + bv_ref[...]

    def bits(a):
        return jax.lax.bitcast_convert_type(
            a.astype(jnp.bfloat16).astype(jnp.float32), jnp.int32)

    def pack(lo, hi):
        return jax.lax.shift_right_logical(bits(lo), 16) | bits(hi)

    td_ref[...] = jnp.concatenate(
        [pack(q, q * k), jax.lax.bitcast_convert_type(v, jnp.int32)], axis=1)
    ts_ref[...] = pack(k, v)


def _prologue(x, W_query, b_query, W_key, b_key, W_value, b_value, interpret=False):
    full = lambda shape: pl.BlockSpec(shape, lambda i: (0, 0))
    return pl.pallas_call(
        _prologue_body,
        grid=(_NB,),
        in_specs=[
            pl.BlockSpec((_BN, D), lambda i: (i, 0)),
            full((D, C)), full((1, C)),
            full((D, C)), full((1, C)),
            full((D, C)), full((1, C)),
        ],
        out_specs=[
            pl.BlockSpec((_BN, 2 * C), lambda i: (i, 0)),
            pl.BlockSpec((_BN, C), lambda i: (i, 0)),
        ],
        out_shape=[
            jax.ShapeDtypeStruct((N, 2 * C), jnp.int32),
            jax.ShapeDtypeStruct((N, C), jnp.int32),
        ],
        interpret=interpret,
    )(x, W_query, b_query.reshape(1, C), W_key, b_key.reshape(1, C),
      W_value, b_value.reshape(1, C))


def _edge_body(ea_ref, gd_ref, gs_ref, wedge_ref,
               wmu1_ref, wmu2_ref, wmu3_ref, bmu_ref,
               wmsg1_ref, wmsg2_ref, wmsg3_ref, bmsg_ref,
               lag_ref, lab_ref, lmg_ref, lmb_ref, z_ref):
    ea = ea_ref[...]
    e = jnp.dot(ea, wedge_ref[...], preferred_element_type=jnp.float32)
    gdi = gd_ref[...]
    gai = gdi[:, 0:C]
    gsi = gs_ref[...]
    f32 = jnp.float32
    bf16 = jnp.bfloat16
    q_i = jax.lax.bitcast_convert_type(jax.lax.shift_left(gai, 16), f32)
    qk_i = jax.lax.bitcast_convert_type(gai & jnp.int32(-65536), f32)
    k_j = jax.lax.bitcast_convert_type(jax.lax.shift_left(gsi, 16), f32)
    v_j = jax.lax.bitcast_convert_type(gsi & jnp.int32(-65536), f32)
    v_i = jax.lax.bitcast_convert_type(gdi[:, C:2 * C], f32)
    scale = 1.0 / math.sqrt(3.0 * C)
    a1 = qk_i * scale
    a2 = (q_i * k_j) * scale
    a3 = (q_i * e) * scale
    inv = 1.0 / (3.0 * C)
    m = (jnp.sum(a1, axis=1, keepdims=True) + jnp.sum(a2, axis=1, keepdims=True)
         + jnp.sum(a3, axis=1, keepdims=True)) * inv
    sq = (jnp.sum(a1 * a1, axis=1, keepdims=True)
          + jnp.sum(a2 * a2, axis=1, keepdims=True)
          + jnp.sum(a3 * a3, axis=1, keepdims=True)) * inv
    rstd = jax.lax.rsqrt(sq - m * m + 1e-5)
    lag = lag_ref[...]
    lab = lab_ref[...]
    g1 = jax.nn.sigmoid((a1 - m) * rstd * lag[:, 0:C] + lab[:, 0:C])
    g2 = jax.nn.sigmoid((a2 - m) * rstd * lag[:, C:2 * C] + lab[:, C:2 * C])
    g3 = jax.nn.sigmoid((a3 - m) * rstd * lag[:, 2 * C:3 * C] + lab[:, 2 * C:3 * C])
    upd = (jnp.dot(v_i.astype(bf16), wmu1_ref[...], preferred_element_type=f32)
           + jnp.dot(v_j.astype(bf16), wmu2_ref[...], preferred_element_type=f32)
           + jnp.dot(e.astype(bf16), wmu3_ref[...], preferred_element_type=f32)
           + bmu_ref[...])
    t1 = (upd[:, 0:C] * g1).astype(bf16)
    t2 = (upd[:, C:2 * C] * g2).astype(bf16)
    t3 = (upd[:, 2 * C:3 * C] * g3).astype(bf16)
    z = (jnp.dot(t1, wmsg1_ref[...], preferred_element_type=f32)
         + jnp.dot(t2, wmsg2_ref[...], preferred_element_type=f32)
         + jnp.dot(t3, wmsg3_ref[...], preferred_element_type=f32)
         + bmsg_ref[...])
    zm = jnp.mean(z, axis=1, keepdims=True)
    zv = jnp.mean(z * z, axis=1, keepdims=True) - zm * zm
    z_ref[...] = (z - zm) * jax.lax.rsqrt(zv + 1e-5) * lmg_ref[...] + lmb_ref[...]


def _edge_stage(edge_attr, G_dv, G_s, W_edge, W_msg_update, b_msg_update,
                W_msg, b_msg, ln_alpha_g, ln_alpha_b, ln_msg_g, ln_msg_b,
                nblocks=_EB, blk0=0, interpret=False):
    full = lambda shape: pl.BlockSpec(shape, lambda i: (0, 0))
    return pl.pallas_call(
        _edge_body,
        grid=(nblocks,),
        in_specs=[
            pl.BlockSpec((_BE, 16), lambda i: (i + blk0, 0)),
            pl.BlockSpec((_BE, 2 * C), lambda i: (i, 0)),
            pl.BlockSpec((_BE, C), lambda i: (i, 0)),
            full((16, C)),
            full((C, 3 * C)), full((C, 3 * C)), full((C, 3 * C)),
            full((1, 3 * C)),
            full((C, C)), full((C, C)), full((C, C)),
            full((1, C)),
            full((1, 3 * C)), full((1, 3 * C)),
            full((1, C)), full((1, C)),
        ],
        out_specs=pl.BlockSpec((_BE, C), lambda i: (i, 0)),
        out_shape=jax.ShapeDtypeStruct((nblocks * _BE, C), jnp.float32),
        interpret=interpret,
    )(edge_attr, G_dv, G_s, W_edge,
      W_msg_update[0:C].astype(jnp.bfloat16),
      W_msg_update[C:2 * C].astype(jnp.bfloat16),
      W_msg_update[2 * C:3 * C].astype(jnp.bfloat16),
      b_msg_update.reshape(1, 3 * C),
      W_msg[0:C].astype(jnp.bfloat16),
      W_msg[C:2 * C].astype(jnp.bfloat16),
      W_msg[2 * C:3 * C].astype(jnp.bfloat16),
      b_msg.reshape(1, C),
      ln_alpha_g.reshape(1, 3 * C), ln_alpha_b.reshape(1, 3 * C),
      ln_msg_g.reshape(1, C), ln_msg_b.reshape(1, C))


def _epi1_body(agg0_ref, agg1_ref, agg2_ref, agg3_ref, wc_ref, bc_ref,
               out1_ref, ssum_ref, ssq_ref):
    i = pl.program_id(0)
    agg = (agg0_ref[...] + agg1_ref[...]) + (agg2_ref[...] + agg3_ref[...])
    o = jnp.dot(agg, wc_ref[...], preferred_element_type=jnp.float32) + bc_ref[...]
    out1_ref[...] = o
    s = jnp.sum(o, axis=0, keepdims=True)
    sq = jnp.sum(o * o, axis=0, keepdims=True)

    @pl.when(i == 0)
    def _():
        ssum_ref[...] = jnp.zeros_like(ssum_ref)
        ssq_ref[...] = jnp.zeros_like(ssq_ref)

    ssum_ref[...] += s
    ssq_ref[...] += sq


def _epi1(agg0, agg1, agg2, agg3, W_concate, b_concate, interpret=False):
    full = lambda shape: pl.BlockSpec(shape, lambda i: (0, 0))
    return pl.pallas_call(
        _epi1_body,
        grid=(_NB,),
        in_specs=[
            pl.BlockSpec((_BN, C), lambda i: (i, 0)),
            pl.BlockSpec((_BN, C), lambda i: (i, 0)),
            pl.BlockSpec((_BN, C), lambda i: (i, 0)),
            pl.BlockSpec((_BN, C), lambda i: (i, 0)),
            full((C, C)), full((1, C)),
        ],
        out_specs=[
            pl.BlockSpec((_BN, C), lambda i: (i, 0)),
            full((1, C)), full((1, C)),
        ],
        out_shape=[
            jax.ShapeDtypeStruct((N, C), jnp.float32),
            jax.ShapeDtypeStruct((1, C), jnp.float32),
            jax.ShapeDtypeStruct((1, C), jnp.float32),
        ],
        interpret=interpret,
    )(agg0, agg1, agg2, agg3, W_concate, b_concate.reshape(1, C))


def _epi2_body(out1_ref, ssum_ref, ssq_ref, x_ref, wskip_ref, bskip_ref,
               bng_ref, bnb_ref, out_ref):
    o = out1_ref[...]
    mean = ssum_ref[...] * (1.0 / N)
    var = ssq_ref[...] * (1.0 / N) - mean * mean
    o = bng_ref[...] * (o - mean) * jax.lax.rsqrt(var + 1e-5) + bnb_ref[...]
    o = o * jax.nn.sigmoid(o)
    skip = jnp.dot(x_ref[...], wskip_ref[...], preferred_element_type=jnp.float32) + bskip_ref[...]
    out_ref[...] = o + skip


def _epi2(out1, ssum, ssq, x, W_skip, b_skip, bn_g, bn_b, interpret=False):
    full = lambda shape: pl.BlockSpec(shape, lambda i: (0, 0))
    return pl.pallas_call(
        _epi2_body,
        grid=(_NB,),
        in_specs=[
            pl.BlockSpec((_BN, C), lambda i: (i, 0)),
            full((1, C)), full((1, C)),
            pl.BlockSpec((_BN, D), lambda i: (i, 0)),
            full((D, C)), full((1, C)),
            full((1, C)), full((1, C)),
        ],
        out_specs=pl.BlockSpec((_BN, C), lambda i: (i, 0)),
        out_shape=jax.ShapeDtypeStruct((N, C), jnp.float32),
        interpret=interpret,
    )(out1, ssum, ssq, x, W_skip, b_skip.reshape(1, C),
      bn_g.reshape(1, C), bn_b.reshape(1, C))


def kernel(x, edge_index, edge_attr, W_query, b_query, W_key, b_key,
           W_value, b_value, W_edge, W_msg_update, b_msg_update, W_msg,
           b_msg, ln_msg_g, ln_msg_b, ln_alpha_g, ln_alpha_b, W_concate,
           b_concate, bn_g, bn_b, W_skip, b_skip):
    src = edge_index[0]
    dst = edge_index[1]
    pad = jnp.zeros((_EPAD - E,), jnp.int32)
    srcp = jnp.concatenate([src, pad])
    dstp = jnp.concatenate([dst, pad])
    TDdv, TS32 = _prologue(x, W_query, b_query, W_key, b_key,
                           W_value, b_value)
    nreal = (E - _EHALF) // _CHUNK
    nblk1 = (E - _EHALF) // _BE
    G_dv0, G_s0 = _sc_gather(srcp, dstp, TDdv, TS32, 0)
    G_dv1, G_s1 = _sc_gather(srcp, dstp, TDdv, TS32, 1)
    z0 = _edge_stage(edge_attr, G_dv0, G_s0, W_edge, W_msg_update,
                     b_msg_update, W_msg, b_msg, ln_alpha_g, ln_alpha_b,
                     ln_msg_g, ln_msg_b, nblocks=_EHALF // _BE, blk0=0)
    z1 = _edge_stage(edge_attr, G_dv1, G_s1, W_edge, W_msg_update,
                     b_msg_update, W_msg, b_msg, ln_alpha_g, ln_alpha_b,
                     ln_msg_g, ln_msg_b, nblocks=nblk1, blk0=_EHALF // _BE)
    p0 = _sc_scatter(z0, dst, 0, _EHALF // _CHUNK)
    p1 = _sc_scatter(z1, dst, _EHALF, nreal)
    out1, ssum, ssq = _epi1(p0[0], p0[1], p1[0], p1[1], W_concate, b_concate)
    return _epi2(out1, ssum, ssq, x, W_skip, b_skip, bn_g, bn_b)


# speedup vs baseline: 1.2692x; 1.1410x over previous
import functools
import math

import jax
import jax.numpy as jnp
from jax import lax
from jax.experimental import pallas as pl
from jax.experimental.pallas import tpu as pltpu
from jax.experimental.pallas import tpu_sc as plsc

N = 10000
E = 160000
D = 128
C = 128

_NB = 10
_BN = N // _NB
_EB = 250
_BE = E // _EB

_NC = 2
_NS = 16
_NW = _NC * _NS
_CHUNK = 128
_NCHUNK = E // _CHUNK
_CPW = -(-_NCHUNK // _NW)
_U = 80
_NU = N // _U
_UPW = -(-_NU // _NS)


_EPAD = _NW * _CPW * _CHUNK
_EPW = _EPAD // _NW


_NSPLIT = 4
_EHALF = _EPAD // _NSPLIT
_EPW_H = _EHALF // _NW
_CPW_H = _EPW_H // _CHUNK


def _make_gather_body(base0):
    def _sc_gather_body(src_ref, dst_ref, td_ref, ts_ref,
                        gd_ref, gs_ref,
                        idx_d, idx_s, rows_d, rows_s, *sems):
        c = lax.axis_index("c")
        s = lax.axis_index("s")
        wid = s * _NC + c
        wbase = wid * _EPW_H
        gd_sems, gs_sems = sems[0:2], sems[2:4]
        wd_sems, ws_sems = sems[4:6], sems[6:8]

        pltpu.sync_copy(dst_ref.at[pl.ds(base0 + wbase, _EPW_H)], idx_d)
        pltpu.sync_copy(src_ref.at[pl.ds(base0 + wbase, _EPW_H)], idx_s)

        def g_descs(b, k):
            i0 = k * _CHUNK
            return (
                pltpu.make_async_copy(
                    td_ref.at[idx_d.at[pl.ds(i0, _CHUNK)]], rows_d.at[b],
                    gd_sems[b]),
                pltpu.make_async_copy(
                    ts_ref.at[idx_s.at[pl.ds(i0, _CHUNK)]], rows_s.at[b],
                    gs_sems[b]),
            )

        def w_descs(b, k):
            off = wbase + k * _CHUNK
            return (
                pltpu.make_async_copy(
                    rows_d.at[b], gd_ref.at[pl.ds(off, _CHUNK)], wd_sems[b]),
                pltpu.make_async_copy(
                    rows_s.at[b], gs_ref.at[pl.ds(off, _CHUNK)], ws_sems[b]),
            )

        for b in (0, 1):
            for dsc in g_descs(b, b):
                dsc.start()

        def body(t, carry):
            for b in (0, 1):
                k = 2 * t + b
                for dsc in g_descs(b, k):
                    dsc.wait()
                wds = w_descs(b, k)
                for dsc in wds:
                    dsc.start()
                for dsc in wds:
                    dsc.wait()

                @pl.when(k + 2 < _CPW_H)
                def _():
                    for dsc in g_descs(b, k + 2):
                        dsc.start()

            return carry

        lax.fori_loop(0, _CPW_H // 2, body, 0)

    return _sc_gather_body


def _sc_gather(src, dst, TDdv, TS32, half):
    mesh = plsc.VectorSubcoreMesh(core_axis_name="c", subcore_axis_name="s")
    return pl.kernel(
        _make_gather_body(half * _EHALF),
        out_type=[
            jax.ShapeDtypeStruct((_EHALF, 2 * C), jnp.int32),
            jax.ShapeDtypeStruct((_EHALF, C), jnp.int32),
        ],
        mesh=mesh,
        scratch_types=[
            pltpu.VMEM((_EPW_H,), jnp.int32),
            pltpu.VMEM((_EPW_H,), jnp.int32),
            pltpu.VMEM((2, _CHUNK, 2 * C), jnp.int32),
            pltpu.VMEM((2, _CHUNK, C), jnp.int32),
        ] + [pltpu.SemaphoreType.DMA] * 8,
    )(src, dst, TDdv, TS32)


def _make_scatter_body(base0, nchunk):
    cpw = -(-nchunk // _NW)

    def _sc_scatter_body(z_ref, dst_ref, out_ref,
                         idx_v, z_v, stage_v, acc_shared, sem):
        c = lax.axis_index("c")
        s = lax.axis_index("s")
        wid = s * _NC + c

        def zbody(i, carry):
            r = i // 8
            l = i % 8
            stage_v[r, pl.ds(l * 16, 16)] = jnp.zeros((16,), jnp.float32)
            return carry

        lax.fori_loop(0, _U * 8, zbody, 0)

        def ubody(j, carry):
            u = s + j * _NS

            @pl.when(u < _NU)
            def _():
                pltpu.sync_copy(stage_v, acc_shared.at[pl.ds(u * _U, _U)])

            return carry

        lax.fori_loop(0, _UPW, ubody, 0)
        plsc.subcore_barrier()

        def body(j, carry):
            ci = wid + j * _NW

            @pl.when(ci < nchunk)
            def _():
                off = ci * _CHUNK
                pltpu.sync_copy(dst_ref.at[pl.ds(base0 + off, _CHUNK)], idx_v)
                pltpu.sync_copy(z_ref.at[pl.ds(off, _CHUNK)], z_v)
                pltpu.sync_copy(z_v, acc_shared.at[idx_v], add=True)

            return carry

        lax.fori_loop(0, cpw, body, 0)
        plsc.subcore_barrier()

        def dbody(j, carry):
            u = s + j * _NS

            @pl.when(u < _NU)
            def _():
                pltpu.sync_copy(acc_shared.at[pl.ds(u * _U, _U)], stage_v)
                pltpu.sync_copy(stage_v, out_ref.at[c, pl.ds(u * _U, _U)])

            return carry

        lax.fori_loop(0, _UPW, dbody, 0)

    return _sc_scatter_body


def _sc_scatter(z, dst, base0, nchunk):
    mesh = plsc.VectorSubcoreMesh(core_axis_name="c", subcore_axis_name="s")
    return pl.kernel(
        _make_scatter_body(base0, nchunk),
        out_type=jax.ShapeDtypeStruct((_NC, N, C), jnp.float32),
        mesh=mesh,
        scratch_types=[
            pltpu.VMEM((_CHUNK,), jnp.int32),
            pltpu.VMEM((_CHUNK, C), jnp.float32),
            pltpu.VMEM((_U, C), jnp.float32),
            pltpu.VMEM_SHARED((N, C), jnp.float32),
            pltpu.SemaphoreType.DMA,
        ],
    )(z, dst)


def _prologue_body(x_ref, wq_ref, bq_ref, wk_ref, bk_ref, wv_ref, bv_ref,
                   td_ref, ts_ref):
    x = x_ref[...]
    q = jnp.dot(x, wq_ref[...], preferred_element_type=jnp.float32) + bq_ref[...]
    k = jnp.dot(x, wk_ref[...], preferred_element_type=jnp.float32) + bk_ref[...]
    v = jnp.dot(x, wv_ref[...], preferred_element_type=jnp.float32) + bv_ref[...]

    def bits(a):
        return jax.lax.bitcast_convert_type(
            a.astype(jnp.bfloat16).astype(jnp.float32), jnp.int32)

    def pack(lo, hi):
        return jax.lax.shift_right_logical(bits(lo), 16) | bits(hi)

    td_ref[...] = jnp.concatenate(
        [pack(q, q * k), jax.lax.bitcast_convert_type(v, jnp.int32)], axis=1)
    ts_ref[...] = pack(k, v)


def _prologue(x, W_query, b_query, W_key, b_key, W_value, b_value, interpret=False):
    full = lambda shape: pl.BlockSpec(shape, lambda i: (0, 0))
    return pl.pallas_call(
        _prologue_body,
        grid=(_NB,),
        in_specs=[
            pl.BlockSpec((_BN, D), lambda i: (i, 0)),
            full((D, C)), full((1, C)),
            full((D, C)), full((1, C)),
            full((D, C)), full((1, C)),
        ],
        out_specs=[
            pl.BlockSpec((_BN, 2 * C), lambda i: (i, 0)),
            pl.BlockSpec((_BN, C), lambda i: (i, 0)),
        ],
        out_shape=[
            jax.ShapeDtypeStruct((N, 2 * C), jnp.int32),
            jax.ShapeDtypeStruct((N, C), jnp.int32),
        ],
        interpret=interpret,
    )(x, W_query, b_query.reshape(1, C), W_key, b_key.reshape(1, C),
      W_value, b_value.reshape(1, C))


def _edge_body(ea_ref, gd_ref, gs_ref, wedge_ref,
               wmu1_ref, wmu2_ref, wmu3_ref, bmu_ref,
               wmsg1_ref, wmsg2_ref, wmsg3_ref, bmsg_ref,
               lag_ref, lab_ref, lmg_ref, lmb_ref, z_ref):
    ea = ea_ref[...]
    e = jnp.dot(ea, wedge_ref[...], preferred_element_type=jnp.float32)
    gdi = gd_ref[...]
    gai = gdi[:, 0:C]
    gsi = gs_ref[...]
    f32 = jnp.float32
    bf16 = jnp.bfloat16
    q_i = jax.lax.bitcast_convert_type(jax.lax.shift_left(gai, 16), f32)
    qk_i = jax.lax.bitcast_convert_type(gai & jnp.int32(-65536), f32)
    k_j = jax.lax.bitcast_convert_type(jax.lax.shift_left(gsi, 16), f32)
    v_j = jax.lax.bitcast_convert_type(gsi & jnp.int32(-65536), f32)
    v_i = jax.lax.bitcast_convert_type(gdi[:, C:2 * C], f32)
    scale = 1.0 / math.sqrt(3.0 * C)
    a1 = qk_i * scale
    a2 = (q_i * k_j) * scale
    a3 = (q_i * e) * scale
    inv = 1.0 / (3.0 * C)
    m = (jnp.sum(a1, axis=1, keepdims=True) + jnp.sum(a2, axis=1, keepdims=True)
         + jnp.sum(a3, axis=1, keepdims=True)) * inv
    sq = (jnp.sum(a1 * a1, axis=1, keepdims=True)
          + jnp.sum(a2 * a2, axis=1, keepdims=True)
          + jnp.sum(a3 * a3, axis=1, keepdims=True)) * inv
    rstd = jax.lax.rsqrt(sq - m * m + 1e-5)
    lag = lag_ref[...]
    lab = lab_ref[...]
    g1 = jax.nn.sigmoid((a1 - m) * rstd * lag[:, 0:C] + lab[:, 0:C])
    g2 = jax.nn.sigmoid((a2 - m) * rstd * lag[:, C:2 * C] + lab[:, C:2 * C])
    g3 = jax.nn.sigmoid((a3 - m) * rstd * lag[:, 2 * C:3 * C] + lab[:, 2 * C:3 * C])
    upd = (jnp.dot(v_i.astype(bf16), wmu1_ref[...], preferred_element_type=f32)
           + jnp.dot(v_j.astype(bf16), wmu2_ref[...], preferred_element_type=f32)
           + jnp.dot(e.astype(bf16), wmu3_ref[...], preferred_element_type=f32)
           + bmu_ref[...])
    t1 = (upd[:, 0:C] * g1).astype(bf16)
    t2 = (upd[:, C:2 * C] * g2).astype(bf16)
    t3 = (upd[:, 2 * C:3 * C] * g3).astype(bf16)
    z = (jnp.dot(t1, wmsg1_ref[...], preferred_element_type=f32)
         + jnp.dot(t2, wmsg2_ref[...], preferred_element_type=f32)
         + jnp.dot(t3, wmsg3_ref[...], preferred_element_type=f32)
         + bmsg_ref[...])
    zm = jnp.mean(z, axis=1, keepdims=True)
    zv = jnp.mean(z * z, axis=1, keepdims=True) - zm * zm
    z_ref[...] = (z - zm) * jax.lax.rsqrt(zv + 1e-5) * lmg_ref[...] + lmb_ref[...]


def _edge_stage(edge_attr, G_dv, G_s, W_edge, W_msg_update, b_msg_update,
                W_msg, b_msg, ln_alpha_g, ln_alpha_b, ln_msg_g, ln_msg_b,
                nblocks=_EB, blk0=0, interpret=False):
    full = lambda shape: pl.BlockSpec(shape, lambda i: (0, 0))
    return pl.pallas_call(
        _edge_body,
        grid=(nblocks,),
        in_specs=[
            pl.BlockSpec((_BE, 16), lambda i: (i + blk0, 0)),
            pl.BlockSpec((_BE, 2 * C), lambda i: (i, 0)),
            pl.BlockSpec((_BE, C), lambda i: (i, 0)),
            full((16, C)),
            full((C, 3 * C)), full((C, 3 * C)), full((C, 3 * C)),
            full((1, 3 * C)),
            full((C, C)), full((C, C)), full((C, C)),
            full((1, C)),
            full((1, 3 * C)), full((1, 3 * C)),
            full((1, C)), full((1, C)),
        ],
        out_specs=pl.BlockSpec((_BE, C), lambda i: (i, 0)),
        out_shape=jax.ShapeDtypeStruct((nblocks * _BE, C), jnp.float32),
        interpret=interpret,
    )(edge_attr, G_dv, G_s, W_edge,
      W_msg_update[0:C].astype(jnp.bfloat16),
      W_msg_update[C:2 * C].astype(jnp.bfloat16),
      W_msg_update[2 * C:3 * C].astype(jnp.bfloat16),
      b_msg_update.reshape(1, 3 * C),
      W_msg[0:C].astype(jnp.bfloat16),
      W_msg[C:2 * C].astype(jnp.bfloat16),
      W_msg[2 * C:3 * C].astype(jnp.bfloat16),
      b_msg.reshape(1, C),
      ln_alpha_g.reshape(1, 3 * C), ln_alpha_b.reshape(1, 3 * C),
      ln_msg_g.reshape(1, C), ln_msg_b.reshape(1, C))


def _epi1_body(agg0_ref, agg1_ref, agg2_ref, agg3_ref, wc_ref, bc_ref,
               out1_ref, ssum_ref, ssq_ref):
    i = pl.program_id(0)
    agg = (agg0_ref[...] + agg1_ref[...]) + (agg2_ref[...] + agg3_ref[...])
    o = jnp.dot(agg, wc_ref[...], preferred_element_type=jnp.float32) + bc_ref[...]
    out1_ref[...] = o
    s = jnp.sum(o, axis=0, keepdims=True)
    sq = jnp.sum(o * o, axis=0, keepdims=True)

    @pl.when(i == 0)
    def _():
        ssum_ref[...] = jnp.zeros_like(ssum_ref)
        ssq_ref[...] = jnp.zeros_like(ssq_ref)

    ssum_ref[...] += s
    ssq_ref[...] += sq


def _epi1(agg0, agg1, agg2, agg3, W_concate, b_concate, interpret=False):
    full = lambda shape: pl.BlockSpec(shape, lambda i: (0, 0))
    return pl.pallas_call(
        _epi1_body,
        grid=(_NB,),
        in_specs=[
            pl.BlockSpec((_BN, C), lambda i: (i, 0)),
            pl.BlockSpec((_BN, C), lambda i: (i, 0)),
            pl.BlockSpec((_BN, C), lambda i: (i, 0)),
            pl.BlockSpec((_BN, C), lambda i: (i, 0)),
            full((C, C)), full((1, C)),
        ],
        out_specs=[
            pl.BlockSpec((_BN, C), lambda i: (i, 0)),
            full((1, C)), full((1, C)),
        ],
        out_shape=[
            jax.ShapeDtypeStruct((N, C), jnp.float32),
            jax.ShapeDtypeStruct((1, C), jnp.float32),
            jax.ShapeDtypeStruct((1, C), jnp.float32),
        ],
        interpret=interpret,
    )(agg0, agg1, agg2, agg3, W_concate, b_concate.reshape(1, C))


def _epi2_body(out1_ref, ssum_ref, ssq_ref, x_ref, wskip_ref, bskip_ref,
               bng_ref, bnb_ref, out_ref):
    o = out1_ref[...]
    mean = ssum_ref[...] * (1.0 / N)
    var = ssq_ref[...] * (1.0 / N) - mean * mean
    o = bng_ref[...] * (o - mean) * jax.lax.rsqrt(var + 1e-5) + bnb_ref[...]
    o = o * jax.nn.sigmoid(o)
    skip = jnp.dot(x_ref[...], wskip_ref[...], preferred_element_type=jnp.float32) + bskip_ref[...]
    out_ref[...] = o + skip


def _epi2(out1, ssum, ssq, x, W_skip, b_skip, bn_g, bn_b, interpret=False):
    full = lambda shape: pl.BlockSpec(shape, lambda i: (0, 0))
    return pl.pallas_call(
        _epi2_body,
        grid=(_NB,),
        in_specs=[
            pl.BlockSpec((_BN, C), lambda i: (i, 0)),
            full((1, C)), full((1, C)),
            pl.BlockSpec((_BN, D), lambda i: (i, 0)),
            full((D, C)), full((1, C)),
            full((1, C)), full((1, C)),
        ],
        out_specs=pl.BlockSpec((_BN, C), lambda i: (i, 0)),
        out_shape=jax.ShapeDtypeStruct((N, C), jnp.float32),
        interpret=interpret,
    )(out1, ssum, ssq, x, W_skip, b_skip.reshape(1, C),
      bn_g.reshape(1, C), bn_b.reshape(1, C))


def kernel(x, edge_index, edge_attr, W_query, b_query, W_key, b_key,
           W_value, b_value, W_edge, W_msg_update, b_msg_update, W_msg,
           b_msg, ln_msg_g, ln_msg_b, ln_alpha_g, ln_alpha_b, W_concate,
           b_concate, bn_g, bn_b, W_skip, b_skip):
    src = edge_index[0]
    dst = edge_index[1]
    pad = jnp.zeros((_EPAD - E,), jnp.int32)
    srcp = jnp.concatenate([src, pad])
    dstp = jnp.concatenate([dst, pad])
    TDdv, TS32 = _prologue(x, W_query, b_query, W_key, b_key,
                           W_value, b_value)
    Gs = [_sc_gather(srcp, dstp, TDdv, TS32, h) for h in range(_NSPLIT)]
    zs = []
    for h in range(_NSPLIT):
        nreal_e = min(E - h * _EHALF, _EHALF)
        zs.append(_edge_stage(
            edge_attr, Gs[h][0], Gs[h][1], W_edge, W_msg_update,
            b_msg_update, W_msg, b_msg, ln_alpha_g, ln_alpha_b,
            ln_msg_g, ln_msg_b, nblocks=nreal_e // _BE,
            blk0=h * (_EHALF // _BE)))
    parts = [_sc_scatter(zs[h], dst, h * _EHALF,
                         min(E - h * _EHALF, _EHALF) // _CHUNK)
             for h in range(_NSPLIT)]
    a0 = sum(p[0] for p in parts[:_NSPLIT // 2])
    a1 = sum(p[1] for p in parts[:_NSPLIT // 2])
    a2 = sum(p[0] for p in parts[_NSPLIT // 2:])
    a3 = sum(p[1] for p in parts[_NSPLIT // 2:])
    out1, ssum, ssq = _epi1(a0, a1, a2, a3, W_concate, b_concate)
    return _epi2(out1, ssum, ssq, x, W_skip, b_skip, bn_g, bn_b)


# speedup vs baseline: 1.2715x; 1.0018x over previous
import math

import jax
import jax.numpy as jnp
from jax import lax
from jax.experimental import pallas as pl
from jax.experimental.pallas import tpu as pltpu
from jax.experimental.pallas import tpu_sc as plsc

N = 10000
E = 160000
D = 128
C = 128

_NB = 10
_BN = N // _NB
_EB = 250
_BE = E // _EB

_NC = 2
_NS = 16
_NW = _NC * _NS
_CHUNK = 128
_U = 80
_NU = N // _U
_UPW = -(-_NU // _NS)

_EPAD = 163840


_NSPLIT = 4
_EHALF = _EPAD // _NSPLIT
_EPW_H = _EHALF // _NW
_CPW_H = _EPW_H // _CHUNK


def _make_gather_body(base0):
    def _sc_gather_body(src_ref, dst_ref, td_ref, ts_ref,
                        gd_ref, gs_ref,
                        idx_d, idx_s, rows_d, rows_s, *sems):
        c = lax.axis_index("c")
        s = lax.axis_index("s")
        wid = s * _NC + c
        wbase = wid * _EPW_H
        gd_sems, gs_sems = sems[0:2], sems[2:4]
        wd_sems, ws_sems = sems[4:6], sems[6:8]

        pltpu.sync_copy(dst_ref.at[pl.ds(base0 + wbase, _EPW_H)], idx_d)
        pltpu.sync_copy(src_ref.at[pl.ds(base0 + wbase, _EPW_H)], idx_s)

        def g_descs(b, k):
            i0 = k * _CHUNK
            return (
                pltpu.make_async_copy(
                    td_ref.at[idx_d.at[pl.ds(i0, _CHUNK)]], rows_d.at[b],
                    gd_sems[b]),
                pltpu.make_async_copy(
                    ts_ref.at[idx_s.at[pl.ds(i0, _CHUNK)]], rows_s.at[b],
                    gs_sems[b]),
            )

        def w_descs(b, k):
            off = wbase + k * _CHUNK
            return (
                pltpu.make_async_copy(
                    rows_d.at[b], gd_ref.at[pl.ds(off, _CHUNK)], wd_sems[b]),
                pltpu.make_async_copy(
                    rows_s.at[b], gs_ref.at[pl.ds(off, _CHUNK)], ws_sems[b]),
            )

        for b in (0, 1):
            for dsc in g_descs(b, b):
                dsc.start()

        def body(t, carry):
            for b in (0, 1):
                k = 2 * t + b
                for dsc in g_descs(b, k):
                    dsc.wait()
                wds = w_descs(b, k)
                for dsc in wds:
                    dsc.start()
                for dsc in wds:
                    dsc.wait()

                @pl.when(k + 2 < _CPW_H)
                def _():
                    for dsc in g_descs(b, k + 2):
                        dsc.start()

            return carry

        lax.fori_loop(0, _CPW_H // 2, body, 0)

    return _sc_gather_body


def _sc_gather(src, dst, TDdv, TS32, half):
    mesh = plsc.VectorSubcoreMesh(core_axis_name="c", subcore_axis_name="s")
    return pl.kernel(
        _make_gather_body(half * _EHALF),
        out_type=[
            jax.ShapeDtypeStruct((_EHALF, 2 * C), jnp.int32),
            jax.ShapeDtypeStruct((_EHALF, C), jnp.int32),
        ],
        mesh=mesh,
        scratch_types=[
            pltpu.VMEM((_EPW_H,), jnp.int32),
            pltpu.VMEM((_EPW_H,), jnp.int32),
            pltpu.VMEM((2, _CHUNK, 2 * C), jnp.int32),
            pltpu.VMEM((2, _CHUNK, C), jnp.int32),
        ] + [pltpu.SemaphoreType.DMA] * 8,
    )(src, dst, TDdv, TS32)


def _make_scatter_body(base0, nchunk):
    cpw = -(-nchunk // _NW)

    def _sc_scatter_body(z_ref, dst_ref, out_ref,
                         idx_v, z_v, stage_v, acc_shared, sem):
        c = lax.axis_index("c")
        s = lax.axis_index("s")
        wid = s * _NC + c

        def zbody(i, carry):
            r = i // 8
            l = i % 8
            stage_v[r, pl.ds(l * 16, 16)] = jnp.zeros((16,), jnp.float32)
            return carry

        lax.fori_loop(0, _U * 8, zbody, 0)

        def ubody(j, carry):
            u = s + j * _NS

            @pl.when(u < _NU)
            def _():
                pltpu.sync_copy(stage_v, acc_shared.at[pl.ds(u * _U, _U)])

            return carry

        lax.fori_loop(0, _UPW, ubody, 0)
        plsc.subcore_barrier()

        def body(j, carry):
            ci = wid + j * _NW

            @pl.when(ci < nchunk)
            def _():
                off = ci * _CHUNK
                pltpu.sync_copy(dst_ref.at[pl.ds(base0 + off, _CHUNK)], idx_v)
                pltpu.sync_copy(z_ref.at[pl.ds(off, _CHUNK)], z_v)
                pltpu.sync_copy(z_v, acc_shared.at[idx_v], add=True)

            return carry

        lax.fori_loop(0, cpw, body, 0)
        plsc.subcore_barrier()

        def dbody(j, carry):
            u = s + j * _NS

            @pl.when(u < _NU)
            def _():
                pltpu.sync_copy(acc_shared.at[pl.ds(u * _U, _U)], stage_v)
                pltpu.sync_copy(stage_v, out_ref.at[c, pl.ds(u * _U, _U)])

            return carry

        lax.fori_loop(0, _UPW, dbody, 0)

    return _sc_scatter_body


def _sc_scatter(z, dst, base0, nchunk):
    mesh = plsc.VectorSubcoreMesh(core_axis_name="c", subcore_axis_name="s")
    return pl.kernel(
        _make_scatter_body(base0, nchunk),
        out_type=jax.ShapeDtypeStruct((_NC, N, C), jnp.float32),
        mesh=mesh,
        scratch_types=[
            pltpu.VMEM((_CHUNK,), jnp.int32),
            pltpu.VMEM((_CHUNK, C), jnp.float32),
            pltpu.VMEM((_U, C), jnp.float32),
            pltpu.VMEM_SHARED((N, C), jnp.float32),
            pltpu.SemaphoreType.DMA,
        ],
    )(z, dst)


def _prologue_body(x_ref, wq_ref, bq_ref, wk_ref, bk_ref, wv_ref, bv_ref,
                   td_ref, ts_ref):
    x = x_ref[...]
    q = jnp.dot(x, wq_ref[...], preferred_element_type=jnp.float32) + bq_ref[...]
    k = jnp.dot(x, wk_ref[...], preferred_element_type=jnp.float32) + bk_ref[...]
    v = jnp.dot(x, wv_ref[...], preferred_element_type=jnp.float32) + bv_ref[...]

    def bits(a):
        return jax.lax.bitcast_convert_type(
            a.astype(jnp.bfloat16).astype(jnp.float32), jnp.int32)

    def pack(lo, hi):
        return jax.lax.shift_right_logical(bits(lo), 16) | bits(hi)

    td_ref[...] = jnp.concatenate(
        [pack(q, q * k), jax.lax.bitcast_convert_type(v, jnp.int32)], axis=1)
    ts_ref[...] = pack(k, v)


def _prologue(x, W_query, b_query, W_key, b_key, W_value, b_value, interpret=False):
    full = lambda shape: pl.BlockSpec(shape, lambda i: (0, 0))
    return pl.pallas_call(
        _prologue_body,
        grid=(_NB,),
        in_specs=[
            pl.BlockSpec((_BN, D), lambda i: (i, 0)),
            full((D, C)), full((1, C)),
            full((D, C)), full((1, C)),
            full((D, C)), full((1, C)),
        ],
        out_specs=[
            pl.BlockSpec((_BN, 2 * C), lambda i: (i, 0)),
            pl.BlockSpec((_BN, C), lambda i: (i, 0)),
        ],
        out_shape=[
            jax.ShapeDtypeStruct((N, 2 * C), jnp.int32),
            jax.ShapeDtypeStruct((N, C), jnp.int32),
        ],
        interpret=interpret,
    )(x, W_query, b_query.reshape(1, C), W_key, b_key.reshape(1, C),
      W_value, b_value.reshape(1, C))


def _edge_body(ea_ref, gd_ref, gs_ref, wedge_ref,
               wmu1_ref, wmu2_ref, wmu3_ref, bmu_ref,
               wmsg1_ref, wmsg2_ref, wmsg3_ref, bmsg_ref,
               lag_ref, lab_ref, lmg_ref, lmb_ref, z_ref):
    ea = ea_ref[...]
    e = jnp.dot(ea, wedge_ref[...], preferred_element_type=jnp.float32)
    gdi = gd_ref[...]
    gai = gdi[:, 0:C]
    gsi = gs_ref[...]
    f32 = jnp.float32
    bf16 = jnp.bfloat16
    q_i = jax.lax.bitcast_convert_type(jax.lax.shift_left(gai, 16), f32)
    qk_i = jax.lax.bitcast_convert_type(gai & jnp.int32(-65536), f32)
    k_j = jax.lax.bitcast_convert_type(jax.lax.shift_left(gsi, 16), f32)
    v_j = jax.lax.bitcast_convert_type(gsi & jnp.int32(-65536), f32)
    v_i = jax.lax.bitcast_convert_type(gdi[:, C:2 * C], f32)
    scale = 1.0 / math.sqrt(3.0 * C)
    a1 = qk_i * scale
    a2 = (q_i * k_j) * scale
    a3 = (q_i * e) * scale
    inv = 1.0 / (3.0 * C)
    m = (jnp.sum(a1, axis=1, keepdims=True) + jnp.sum(a2, axis=1, keepdims=True)
         + jnp.sum(a3, axis=1, keepdims=True)) * inv
    sq = (jnp.sum(a1 * a1, axis=1, keepdims=True)
          + jnp.sum(a2 * a2, axis=1, keepdims=True)
          + jnp.sum(a3 * a3, axis=1, keepdims=True)) * inv
    rstd = jax.lax.rsqrt(sq - m * m + 1e-5)
    lag = lag_ref[...]
    lab = lab_ref[...]
    g1 = jax.nn.sigmoid((a1 - m) * rstd * lag[:, 0:C] + lab[:, 0:C])
    g2 = jax.nn.sigmoid((a2 - m) * rstd * lag[:, C:2 * C] + lab[:, C:2 * C])
    g3 = jax.nn.sigmoid((a3 - m) * rstd * lag[:, 2 * C:3 * C] + lab[:, 2 * C:3 * C])
    upd = (jnp.dot(v_i.astype(bf16), wmu1_ref[...], preferred_element_type=f32)
           + jnp.dot(v_j.astype(bf16), wmu2_ref[...], preferred_element_type=f32)
           + jnp.dot(e.astype(bf16), wmu3_ref[...], preferred_element_type=f32)
           + bmu_ref[...])
    t1 = (upd[:, 0:C] * g1).astype(bf16)
    t2 = (upd[:, C:2 * C] * g2).astype(bf16)
    t3 = (upd[:, 2 * C:3 * C] * g3).astype(bf16)
    z = (jnp.dot(t1, wmsg1_ref[...], preferred_element_type=f32)
         + jnp.dot(t2, wmsg2_ref[...], preferred_element_type=f32)
         + jnp.dot(t3, wmsg3_ref[...], preferred_element_type=f32)
         + bmsg_ref[...])
    zm = jnp.mean(z, axis=1, keepdims=True)
    zv = jnp.mean(z * z, axis=1, keepdims=True) - zm * zm
    z_ref[...] = (z - zm) * jax.lax.rsqrt(zv + 1e-5) * lmg_ref[...] + lmb_ref[...]


def _edge_stage(edge_attr, G_dv, G_s, W_edge, W_msg_update, b_msg_update,
                W_msg, b_msg, ln_alpha_g, ln_alpha_b, ln_msg_g, ln_msg_b,
                nblocks=_EB, blk0=0, interpret=False):
    full = lambda shape: pl.BlockSpec(shape, lambda i: (0, 0))
    return pl.pallas_call(
        _edge_body,
        grid=(nblocks,),
        in_specs=[
            pl.BlockSpec((_BE, 16), lambda i: (i + blk0, 0)),
            pl.BlockSpec((_BE, 2 * C), lambda i: (i, 0)),
            pl.BlockSpec((_BE, C), lambda i: (i, 0)),
            full((16, C)),
            full((C, 3 * C)), full((C, 3 * C)), full((C, 3 * C)),
            full((1, 3 * C)),
            full((C, C)), full((C, C)), full((C, C)),
            full((1, C)),
            full((1, 3 * C)), full((1, 3 * C)),
            full((1, C)), full((1, C)),
        ],
        out_specs=pl.BlockSpec((_BE, C), lambda i: (i, 0)),
        out_shape=jax.ShapeDtypeStruct((nblocks * _BE, C), jnp.float32),
        interpret=interpret,
    )(edge_attr, G_dv, G_s, W_edge,
      W_msg_update[0:C].astype(jnp.bfloat16),
      W_msg_update[C:2 * C].astype(jnp.bfloat16),
      W_msg_update[2 * C:3 * C].astype(jnp.bfloat16),
      b_msg_update.reshape(1, 3 * C),
      W_msg[0:C].astype(jnp.bfloat16),
      W_msg[C:2 * C].astype(jnp.bfloat16),
      W_msg[2 * C:3 * C].astype(jnp.bfloat16),
      b_msg.reshape(1, C),
      ln_alpha_g.reshape(1, 3 * C), ln_alpha_b.reshape(1, 3 * C),
      ln_msg_g.reshape(1, C), ln_msg_b.reshape(1, C))


def _epi1_body(agg0_ref, agg1_ref, agg2_ref, agg3_ref, wc_ref, bc_ref,
               out1_ref, ssum_ref, ssq_ref):
    i = pl.program_id(0)
    agg = (agg0_ref[...] + agg1_ref[...]) + (agg2_ref[...] + agg3_ref[...])
    o = jnp.dot(agg, wc_ref[...], preferred_element_type=jnp.float32) + bc_ref[...]
    out1_ref[...] = o
    s = jnp.sum(o, axis=0, keepdims=True)
    sq = jnp.sum(o * o, axis=0, keepdims=True)

    @pl.when(i == 0)
    def _():
        ssum_ref[...] = jnp.zeros_like(ssum_ref)
        ssq_ref[...] = jnp.zeros_like(ssq_ref)

    ssum_ref[...] += s
    ssq_ref[...] += sq


def _epi1(agg0, agg1, agg2, agg3, W_concate, b_concate, interpret=False):
    full = lambda shape: pl.BlockSpec(shape, lambda i: (0, 0))
    return pl.pallas_call(
        _epi1_body,
        grid=(_NB,),
        in_specs=[
            pl.BlockSpec((_BN, C), lambda i: (i, 0)),
            pl.BlockSpec((_BN, C), lambda i: (i, 0)),
            pl.BlockSpec((_BN, C), lambda i: (i, 0)),
            pl.BlockSpec((_BN, C), lambda i: (i, 0)),
            full((C, C)), full((1, C)),
        ],
        out_specs=[
            pl.BlockSpec((_BN, C), lambda i: (i, 0)),
            full((1, C)), full((1, C)),
        ],
        out_shape=[
            jax.ShapeDtypeStruct((N, C), jnp.float32),
            jax.ShapeDtypeStruct((1, C), jnp.float32),
            jax.ShapeDtypeStruct((1, C), jnp.float32),
        ],
        interpret=interpret,
    )(agg0, agg1, agg2, agg3, W_concate, b_concate.reshape(1, C))


def _epi2_body(out1_ref, ssum_ref, ssq_ref, x_ref, wskip_ref, bskip_ref,
               bng_ref, bnb_ref, out_ref):
    o = out1_ref[...]
    mean = ssum_ref[...] * (1.0 / N)
    var = ssq_ref[...] * (1.0 / N) - mean * mean
    o = bng_ref[...] * (o - mean) * jax.lax.rsqrt(var + 1e-5) + bnb_ref[...]
    o = o * jax.nn.sigmoid(o)
    skip = jnp.dot(x_ref[...], wskip_ref[...], preferred_element_type=jnp.float32) + bskip_ref[...]
    out_ref[...] = o + skip


def _epi2(out1, ssum, ssq, x, W_skip, b_skip, bn_g, bn_b, interpret=False):
    full = lambda shape: pl.BlockSpec(shape, lambda i: (0, 0))
    return pl.pallas_call(
        _epi2_body,
        grid=(_NB,),
        in_specs=[
            pl.BlockSpec((_BN, C), lambda i: (i, 0)),
            full((1, C)), full((1, C)),
            pl.BlockSpec((_BN, D), lambda i: (i, 0)),
            full((D, C)), full((1, C)),
            full((1, C)), full((1, C)),
        ],
        out_specs=pl.BlockSpec((_BN, C), lambda i: (i, 0)),
        out_shape=jax.ShapeDtypeStruct((N, C), jnp.float32),
        interpret=interpret,
    )(out1, ssum, ssq, x, W_skip, b_skip.reshape(1, C),
      bn_g.reshape(1, C), bn_b.reshape(1, C))


def kernel(x, edge_index, edge_attr, W_query, b_query, W_key, b_key,
           W_value, b_value, W_edge, W_msg_update, b_msg_update, W_msg,
           b_msg, ln_msg_g, ln_msg_b, ln_alpha_g, ln_alpha_b, W_concate,
           b_concate, bn_g, bn_b, W_skip, b_skip):
    src = edge_index[0]
    dst = edge_index[1]
    pad = jnp.zeros((_EPAD - E,), jnp.int32)
    srcp = jnp.concatenate([src, pad])
    dstp = jnp.concatenate([dst, pad])
    TDdv, TS32 = _prologue(x, W_query, b_query, W_key, b_key,
                           W_value, b_value)
    Gs = [_sc_gather(srcp, dstp, TDdv, TS32, h) for h in range(_NSPLIT)]
    zs = []
    for h in range(_NSPLIT):
        nreal_e = min(E - h * _EHALF, _EHALF)
        zs.append(_edge_stage(
            edge_attr, Gs[h][0], Gs[h][1], W_edge, W_msg_update,
            b_msg_update, W_msg, b_msg, ln_alpha_g, ln_alpha_b,
            ln_msg_g, ln_msg_b, nblocks=nreal_e // _BE,
            blk0=h * (_EHALF // _BE)))
    parts = [_sc_scatter(zs[h], dst, h * _EHALF,
                         min(E - h * _EHALF, _EHALF) // _CHUNK)
             for h in range(_NSPLIT)]
    a0 = sum(p[0] for p in parts[:_NSPLIT // 2])
    a1 = sum(p[1] for p in parts[:_NSPLIT // 2])
    a2 = sum(p[0] for p in parts[_NSPLIT // 2:])
    a3 = sum(p[1] for p in parts[_NSPLIT // 2:])
    out1, ssum, ssq = _epi1(a0, a1, a2, a3, W_concate, b_concate)
    return _epi2(out1, ssum, ssq, x, W_skip, b_skip, bn_g, bn_b)
